# Initial kernel scaffold; baseline (speedup 1.0000x reference)
import jax
import jax.numpy as jnp
from jax.experimental import pallas as pl


def kernel(x, keys_w, values, expert_sel):
    raise NotImplementedError("write your pallas kernel here")


# speedup vs baseline: 55.6761x; 55.6761x over previous
import functools

import jax
import jax.numpy as jnp
from jax.experimental import pallas as pl
from jax.experimental.pallas import tpu as pltpu

DMODEL = 1024
N_EXPERTS = 16
EXPERT_SIZE = 128
N_HEADS = 2
EH = N_EXPERTS * EXPERT_SIZE

TB = 512


def _moe_block(x_ref, est_ref, k_ref, v_ref, out_ref):
    x = x_ref[...]
    sel = jnp.dot(x, est_ref[...], preferred_element_type=jnp.float32)

    iota = jax.lax.broadcasted_iota(jnp.int32, (TB, N_EXPERTS), 1)
    m1 = jnp.max(sel, axis=1, keepdims=True)
    idx1 = jnp.min(jnp.where(sel == m1, iota, N_EXPERTS), axis=1, keepdims=True)
    sel2 = jnp.where(iota == idx1, -jnp.inf, sel)
    m2 = jnp.max(sel2, axis=1, keepdims=True)
    idx2 = jnp.min(jnp.where(sel2 == m2, iota, N_EXPERTS), axis=1, keepdims=True)
    mask = (iota == idx1) | (iota == idx2)
    g = jnp.where(mask, sel, -1e30)

    g_big = jnp.reshape(
        jnp.broadcast_to(g[:, :, None], (TB, N_EXPERTS, EXPERT_SIZE)),
        (TB, EH),
    )

    xb = x.astype(jnp.bfloat16)
    h = jnp.dot(xb, k_ref[...], preferred_element_type=jnp.float32)
    h = jnp.maximum(h + g_big, 0.0)
    out_ref[...] = jnp.dot(h.astype(jnp.bfloat16), v_ref[...],
                           preferred_element_type=jnp.float32)


@jax.jit
def kernel(x, keys_w, values, expert_sel):
    n = x.shape[0]
    k_all = keys_w.transpose(1, 0, 2).reshape(DMODEL, EH).astype(jnp.bfloat16)
    v_all = values.reshape(EH, DMODEL).astype(jnp.bfloat16)
    est = expert_sel.T

    grid = (n // TB,)
    return pl.pallas_call(
        _moe_block,
        grid=grid,
        in_specs=[
            pl.BlockSpec((TB, DMODEL), lambda i: (i, 0)),
            pl.BlockSpec((DMODEL, N_EXPERTS), lambda i: (0, 0)),
            pl.BlockSpec((DMODEL, EH), lambda i: (0, 0)),
            pl.BlockSpec((EH, DMODEL), lambda i: (0, 0)),
        ],
        out_specs=pl.BlockSpec((TB, DMODEL), lambda i: (i, 0)),
        out_shape=jax.ShapeDtypeStruct((n, DMODEL), jnp.float32),
    )(x, est, k_all, v_all)
